# baseline (speedup 1.0000x reference)
import functools

import jax
import jax.numpy as jnp
from jax import lax
from jax.experimental import pallas as pl
from jax.experimental.pallas import tpu as pltpu
from jax.experimental.pallas import tpu_sc as plsc

_N = 10000
_E = 160000
_NTILE = 16
_LANES = 16
_EROWS = _E // 128
_ROWS = 80
_GROUPS = _ROWS // 16
_FULL_TILES = _EROWS // _ROWS
_TAIL_ROWS = _EROWS - _FULL_TILES * _ROWS
_NP = 10240
_SLICE = _NP // _NTILE
_TRASH = _NP - 1
_ROUNDS = 4


def _rsqrt16(x):
    i = lax.bitcast_convert_type(x, jnp.int32)
    i = jnp.int32(0x5F3759DF) - lax.shift_right_arithmetic(i, 1)
    y = lax.bitcast_convert_type(i, jnp.float32)
    for _ in range(3):
        y = y * (1.5 - 0.5 * x * y * y)
    return y


def _sc_body(ei_hbm, out_hbm, src_v, dst_v, w_v, vals_v, ubuf_v, slice_v,
             a_sh, b_sh, sem):
    s = lax.axis_index("s")
    c = lax.axis_index("c")
    base = s * _SLICE

    trash16 = jnp.full((_LANES,), _TRASH, jnp.int32)
    nrows = jnp.where(s == _FULL_TILES, _TAIL_ROWS, _ROWS)
    eoff = s * (_ROWS * 128)

    def _stage_row(j, _):
        pltpu.async_copy(ei_hbm.at[0, pl.ds(eoff + j * 128, 128)],
                         src_v.at[j], sem)
        pltpu.async_copy(ei_hbm.at[1, pl.ds(eoff + j * 128, 128)],
                         dst_v.at[j], sem)
        return _

    def _stage_drain(j, _):
        pltpu.make_async_copy(ei_hbm.at[0, pl.ds(eoff + j * 128, 128)],
                              src_v.at[j], sem).wait()
        pltpu.make_async_copy(ei_hbm.at[1, pl.ds(eoff + j * 128, 128)],
                              dst_v.at[j], sem).wait()
        return _

    def _fill_trash(j, _):
        for k in range(8):
            src_v[j, pl.ds(k * _LANES, _LANES)] = trash16
            dst_v[j, pl.ds(k * _LANES, _LANES)] = trash16
        return _

    lax.fori_loop(0, nrows, _stage_row, None)
    lax.fori_loop(nrows, _ROWS, _fill_trash, None)
    lax.fori_loop(0, nrows, _stage_drain, None)

    zero16 = jnp.zeros((_LANES,), jnp.float32)
    one16 = jnp.full((_LANES,), 1.0, jnp.float32)

    def _zero_slice(i, _):
        slice_v[pl.ds(i * _LANES, _LANES)] = zero16
        return _

    def _drain_sweep(dst_sh):
        pltpu.make_async_copy(ubuf_v, dst_sh, sem).wait()

    lax.fori_loop(0, _SLICE // _LANES, _zero_slice, None)
    pltpu.sync_copy(slice_v, a_sh.at[pl.ds(base, _SLICE)])

    def _fill_ones(j, _):
        for k in range(8):
            vals_v[j, pl.ds(k * _LANES, _LANES)] = one16
        return _

    lax.fori_loop(0, _ROWS, _fill_ones, None)
    plsc.subcore_barrier()

    def _deg_row(j, _):
        pltpu.async_copy(vals_v.at[j], a_sh.at[dst_v.at[j]], sem, add=True)
        return _

    lax.fori_loop(0, _ROWS, _deg_row, None)
    _drain_sweep(a_sh)
    plsc.subcore_barrier()

    pltpu.sync_copy(a_sh.at[pl.ds(base, _SLICE)], slice_v)

    def _norm_step(i, _):
        d = slice_v[pl.ds(i * _LANES, _LANES)]
        slice_v[pl.ds(i * _LANES, _LANES)] = _rsqrt16(jnp.maximum(d, 1.0))
        return _

    lax.fori_loop(0, _SLICE // _LANES, _norm_step, None)
    pltpu.sync_copy(slice_v, b_sh.at[pl.ds(base, _SLICE)])
    plsc.subcore_barrier()

    pltpu.sync_copy(b_sh, ubuf_v)
    plsc.subcore_barrier()

    inv_n = jnp.float32(1.0 / _N)

    def _w_row(j, _):
        for k in range(8):
            sl = pl.ds(k * _LANES, _LANES)
            ns = plsc.load_gather(ubuf_v, [src_v[j, sl]])
            nd = plsc.load_gather(ubuf_v, [dst_v[j, sl]])
            w = ns * nd
            w_v[j, sl] = w
            vals_v[j, sl] = w * inv_n
        return _

    lax.fori_loop(0, _ROWS, _w_row, None)
    lax.fori_loop(0, _SLICE // _LANES, _zero_slice, None)
    pltpu.sync_copy(slice_v, b_sh.at[pl.ds(base, _SLICE)])
    plsc.subcore_barrier()

    def _r0_row(j, _):
        pltpu.async_copy(vals_v.at[j], b_sh.at[src_v.at[j]], sem, add=True)
        return _

    lax.fori_loop(0, _ROWS, _r0_row, None)
    _drain_sweep(b_sh)
    plsc.subcore_barrier()

    for r in range(1, _ROUNDS):
        rd, wr = (b_sh, a_sh) if r % 2 == 1 else (a_sh, b_sh)
        pltpu.sync_copy(rd, ubuf_v)
        lax.fori_loop(0, _SLICE // _LANES, _zero_slice, None)
        pltpu.sync_copy(slice_v, wr.at[pl.ds(base, _SLICE)])
        plsc.subcore_barrier()

        def _mp_row(j, _):
            for k in range(8):
                sl = pl.ds(k * _LANES, _LANES)
                uv = plsc.load_gather(ubuf_v, [dst_v[j, sl]])
                vals_v[j, sl] = uv * w_v[j, sl]
            pltpu.async_copy(vals_v.at[j], wr.at[src_v.at[j]], sem, add=True)
            return _

        lax.fori_loop(0, _ROWS, _mp_row, None)
        _drain_sweep(wr)
        plsc.subcore_barrier()

    pltpu.sync_copy(a_sh.at[pl.ds(base, _SLICE)], slice_v)

    @pl.when(c == 0)
    def _():
        pltpu.sync_copy(slice_v, out_hbm.at[0, pl.ds(base, _SLICE)])


_sc_u = functools.partial(
    pl.kernel,
    out_type=jax.ShapeDtypeStruct((1, _NP), jnp.float32),
    mesh=plsc.VectorSubcoreMesh(core_axis_name="c", subcore_axis_name="s"),
    compiler_params=pltpu.CompilerParams(needs_layout_passes=False),
    scratch_types=[
        pltpu.VMEM((_ROWS, 128), jnp.int32),
        pltpu.VMEM((_ROWS, 128), jnp.int32),
        pltpu.VMEM((_ROWS, 128), jnp.float32),
        pltpu.VMEM((_ROWS, 128), jnp.float32),
        pltpu.VMEM((_NP,), jnp.float32),
        pltpu.VMEM((_SLICE,), jnp.float32),
        pltpu.VMEM_SHARED((_NP,), jnp.float32),
        pltpu.VMEM_SHARED((_NP,), jnp.float32),
        pltpu.SemaphoreType.DMA,
    ],
)(_sc_body)


def _tc_body(u_ref, feat_ref, wemb_ref, bemb_ref, w1_ref, b1_ref, w2_ref,
             b2_ref, wr_ref, br_ref, out_ref):
    u = u_ref[...]
    dn = (((1,), (1,)), ((), ()))
    hi = lax.Precision.HIGHEST
    v1 = lax.dot_general(u, feat_ref[...], (((1,), (0,)), ((), ())),
                         preferred_element_type=jnp.float32, precision=hi)
    su = jnp.sum(u)
    hg = lax.dot_general(v1, wemb_ref[...], dn,
                         preferred_element_type=jnp.float32,
                         precision=hi) + su * bemb_ref[...].reshape(1, -1)
    hg = lax.dot_general(hg, w1_ref[...], dn,
                         preferred_element_type=jnp.float32,
                         precision=hi) + b1_ref[...].reshape(1, -1)
    hg = lax.dot_general(hg, w2_ref[...], dn,
                         preferred_element_type=jnp.float32,
                         precision=hi) + b2_ref[...].reshape(1, -1)
    out_ref[...] = lax.dot_general(hg, wr_ref[...], dn,
                                   preferred_element_type=jnp.float32,
                                   precision=hi) + br_ref[...].reshape(1, -1)


def kernel(feat, edge_index, e, snorm_n, snorm_e, W_emb, b_emb, W1, b1, W2,
           b2, Wr, br):
    u = _sc_u(edge_index)[:, :_N]
    nc = Wr.shape[0]
    out = pl.pallas_call(
        _tc_body,
        out_shape=jax.ShapeDtypeStruct((1, nc), jnp.float32),
    )(u, feat, W_emb, b_emb, W1, b1, W2, b2, Wr, br)
    return out


# speedup vs baseline: 1.1808x; 1.1808x over previous
import functools

import jax
import jax.numpy as jnp
from jax import lax
from jax.experimental import pallas as pl
from jax.experimental.pallas import tpu as pltpu
from jax.experimental.pallas import tpu_sc as plsc

_N = 10000
_E = 160000
_NTILE = 16
_LANES = 16
_EROWS = _E // 128
_ROWS = 80
_FULL_TILES = _EROWS // _ROWS
_TAIL_ROWS = _EROWS - _FULL_TILES * _ROWS
_NP = 10240
_SLICE = _NP // _NTILE
_TRASH = _NP - 1
_ROUNDS = 4


def _rsqrt16(x):
    i = lax.bitcast_convert_type(x, jnp.int32)
    i = jnp.int32(0x5F3759DF) - lax.shift_right_arithmetic(i, 1)
    y = lax.bitcast_convert_type(i, jnp.float32)
    for _ in range(3):
        y = y * (1.5 - 0.5 * x * y * y)
    return y


def _sc_body(ei_hbm, out_hbm, src_v, dst_v, w_v, vals_v, ubuf_v, slice_v,
             a_sh, b_sh, sem):
    s = lax.axis_index("s")
    c = lax.axis_index("c")
    base = s * _SLICE

    trash16 = jnp.full((_LANES,), _TRASH, jnp.int32)
    nrows = jnp.where(s == _FULL_TILES, _TAIL_ROWS, _ROWS)
    eoff = s * (_ROWS * 128)

    def _stage_row(j, _):
        pltpu.async_copy(ei_hbm.at[0, pl.ds(eoff + j * 128, 128)],
                         src_v.at[j], sem)
        pltpu.async_copy(ei_hbm.at[1, pl.ds(eoff + j * 128, 128)],
                         dst_v.at[j], sem)
        return _

    def _stage_drain(j, _):
        pltpu.make_async_copy(ei_hbm.at[0, pl.ds(eoff + j * 128, 128)],
                              src_v.at[j], sem).wait()
        pltpu.make_async_copy(ei_hbm.at[1, pl.ds(eoff + j * 128, 128)],
                              dst_v.at[j], sem).wait()
        return _

    def _fill_trash(j, _):
        for k in range(8):
            src_v[j, pl.ds(k * _LANES, _LANES)] = trash16
            dst_v[j, pl.ds(k * _LANES, _LANES)] = trash16
        return _

    lax.fori_loop(0, nrows, _stage_row, None)
    lax.fori_loop(nrows, _ROWS, _fill_trash, None)
    lax.fori_loop(0, nrows, _stage_drain, None)

    zero16 = jnp.zeros((_LANES,), jnp.float32)
    one16 = jnp.full((_LANES,), 1.0, jnp.float32)

    def _zero_slice(i, _):
        slice_v[pl.ds(i * _LANES, _LANES)] = zero16
        return _

    def _drain_sweep(dst_sh):
        pltpu.make_async_copy(ubuf_v, dst_sh, sem).wait()

    lax.fori_loop(0, _SLICE // _LANES, _zero_slice, None)
    pltpu.sync_copy(slice_v, a_sh.at[pl.ds(base, _SLICE)])

    def _fill_ones(j, _):
        for k in range(8):
            vals_v[j, pl.ds(k * _LANES, _LANES)] = one16
        return _

    lax.fori_loop(0, _ROWS, _fill_ones, None)
    plsc.subcore_barrier()

    def _deg_row(j, _):
        pltpu.async_copy(vals_v.at[j], a_sh.at[dst_v.at[j]], sem, add=True)
        return _

    lax.fori_loop(0, _ROWS, _deg_row, None)
    _drain_sweep(a_sh)
    plsc.subcore_barrier()

    pltpu.sync_copy(a_sh.at[pl.ds(base, _SLICE)], slice_v)

    def _norm_step(i, _):
        d = slice_v[pl.ds(i * _LANES, _LANES)]
        slice_v[pl.ds(i * _LANES, _LANES)] = _rsqrt16(jnp.maximum(d, 1.0))
        return _

    lax.fori_loop(0, _SLICE // _LANES, _norm_step, None)
    pltpu.sync_copy(slice_v, b_sh.at[pl.ds(base, _SLICE)])
    plsc.subcore_barrier()

    pltpu.sync_copy(b_sh, ubuf_v)
    plsc.subcore_barrier()
    lax.fori_loop(0, _SLICE // _LANES, _zero_slice, None)
    pltpu.sync_copy(slice_v, b_sh.at[pl.ds(base, _SLICE)])
    plsc.subcore_barrier()

    inv_n = jnp.float32(1.0 / _N)

    def _w_row(j, _):
        for k in range(8):
            sl = pl.ds(k * _LANES, _LANES)
            ns = plsc.load_gather(ubuf_v, [src_v[j, sl]])
            nd = plsc.load_gather(ubuf_v, [dst_v[j, sl]])
            w = ns * nd
            w_v[j, sl] = w
            vals_v[j, sl] = w * inv_n
        pltpu.async_copy(vals_v.at[j], b_sh.at[src_v.at[j]], sem, add=True)
        return _

    lax.fori_loop(0, _ROWS, _w_row, None)
    _drain_sweep(b_sh)
    plsc.subcore_barrier()

    for r in range(1, _ROUNDS):
        rd, wr = (b_sh, a_sh) if r % 2 == 1 else (a_sh, b_sh)
        pltpu.sync_copy(rd, ubuf_v)
        lax.fori_loop(0, _SLICE // _LANES, _zero_slice, None)
        pltpu.sync_copy(slice_v, wr.at[pl.ds(base, _SLICE)])
        plsc.subcore_barrier()

        def _mp_row(j, _):
            for k in range(8):
                sl = pl.ds(k * _LANES, _LANES)
                uv = plsc.load_gather(ubuf_v, [dst_v[j, sl]])
                vals_v[j, sl] = uv * w_v[j, sl]
            pltpu.async_copy(vals_v.at[j], wr.at[src_v.at[j]], sem, add=True)
            return _

        lax.fori_loop(0, _ROWS, _mp_row, None)
        _drain_sweep(wr)
        plsc.subcore_barrier()

    pltpu.sync_copy(a_sh.at[pl.ds(base, _SLICE)], slice_v)

    @pl.when(c == 0)
    def _():
        pltpu.sync_copy(slice_v, out_hbm.at[0, pl.ds(base, _SLICE)])


_sc_u = functools.partial(
    pl.kernel,
    out_type=jax.ShapeDtypeStruct((1, _NP), jnp.float32),
    mesh=plsc.VectorSubcoreMesh(core_axis_name="c", subcore_axis_name="s"),
    compiler_params=pltpu.CompilerParams(needs_layout_passes=False),
    scratch_types=[
        pltpu.VMEM((_ROWS, 128), jnp.int32),
        pltpu.VMEM((_ROWS, 128), jnp.int32),
        pltpu.VMEM((_ROWS, 128), jnp.float32),
        pltpu.VMEM((_ROWS, 128), jnp.float32),
        pltpu.VMEM((_NP,), jnp.float32),
        pltpu.VMEM((_SLICE,), jnp.float32),
        pltpu.VMEM_SHARED((_NP,), jnp.float32),
        pltpu.VMEM_SHARED((_NP,), jnp.float32),
        pltpu.SemaphoreType.DMA,
    ],
)(_sc_body)


def _tc_body(u_ref, feat_ref, wemb_ref, bemb_ref, w1_ref, b1_ref, w2_ref,
             b2_ref, wr_ref, br_ref, out_ref):
    u = u_ref[...]
    dn = (((1,), (1,)), ((), ()))
    hi = lax.Precision.DEFAULT
    v1 = lax.dot_general(u, feat_ref[...], (((1,), (0,)), ((), ())),
                         preferred_element_type=jnp.float32, precision=hi)
    su = jnp.sum(u)
    hg = lax.dot_general(v1, wemb_ref[...], dn,
                         preferred_element_type=jnp.float32,
                         precision=hi) + su * bemb_ref[...].reshape(1, -1)
    hg = lax.dot_general(hg, w1_ref[...], dn,
                         preferred_element_type=jnp.float32,
                         precision=hi) + b1_ref[...].reshape(1, -1)
    hg = lax.dot_general(hg, w2_ref[...], dn,
                         preferred_element_type=jnp.float32,
                         precision=hi) + b2_ref[...].reshape(1, -1)
    out_ref[...] = lax.dot_general(hg, wr_ref[...], dn,
                                   preferred_element_type=jnp.float32,
                                   precision=hi) + br_ref[...].reshape(1, -1)


def kernel(feat, edge_index, e, snorm_n, snorm_e, W_emb, b_emb, W1, b1, W2,
           b2, Wr, br):
    u = _sc_u(edge_index)[:, :_N]
    nc = Wr.shape[0]
    out = pl.pallas_call(
        _tc_body,
        out_shape=jax.ShapeDtypeStruct((1, nc), jnp.float32),
    )(u, feat, W_emb, b_emb, W1, b1, W2, b2, Wr, br)
    return out


# speedup vs baseline: 1.1862x; 1.0046x over previous
import functools

import jax
import jax.numpy as jnp
from jax import lax
from jax.experimental import pallas as pl
from jax.experimental.pallas import tpu as pltpu
from jax.experimental.pallas import tpu_sc as plsc

_N = 10000
_E = 160000
_NTILE = 16
_LANES = 16
_EROWS = _E // 128
_ROWS = 80
_FULL_TILES = _EROWS // _ROWS
_TAIL_ROWS = _EROWS - _FULL_TILES * _ROWS
_NP = 10240
_SLICE = _NP // _NTILE
_TRASH = _NP - 1
_ROUNDS = 4


def _rsqrt16(x):
    i = lax.bitcast_convert_type(x, jnp.int32)
    i = jnp.int32(0x5F3759DF) - lax.shift_right_arithmetic(i, 1)
    y = lax.bitcast_convert_type(i, jnp.float32)
    for _ in range(3):
        y = y * (1.5 - 0.5 * x * y * y)
    return y


def _sc_body(ei_hbm, out_hbm, src_v, dst_v, w_v, vals_v, ubuf_v, slice_v,
             a_sh, b_sh, sem):
    s = lax.axis_index("s")
    c = lax.axis_index("c")
    base = s * _SLICE

    trash16 = jnp.full((_LANES,), _TRASH, jnp.int32)
    nrows = jnp.where(s == _FULL_TILES, _TAIL_ROWS, _ROWS)
    eoff = s * (_ROWS * 128)

    def _stage_row(j, _):
        pltpu.async_copy(ei_hbm.at[0, pl.ds(eoff + j * 128, 128)],
                         src_v.at[j], sem)
        pltpu.async_copy(ei_hbm.at[1, pl.ds(eoff + j * 128, 128)],
                         dst_v.at[j], sem)
        return _

    def _stage_drain(j, _):
        pltpu.make_async_copy(ei_hbm.at[0, pl.ds(eoff + j * 128, 128)],
                              src_v.at[j], sem).wait()
        pltpu.make_async_copy(ei_hbm.at[1, pl.ds(eoff + j * 128, 128)],
                              dst_v.at[j], sem).wait()
        return _

    def _fill_trash(j, _):
        for k in range(8):
            src_v[j, pl.ds(k * _LANES, _LANES)] = trash16
            dst_v[j, pl.ds(k * _LANES, _LANES)] = trash16
        return _

    lax.fori_loop(0, nrows, _stage_row, None)
    lax.fori_loop(nrows, _ROWS, _fill_trash, None)

    zero16 = jnp.zeros((_LANES,), jnp.float32)
    one16 = jnp.full((_LANES,), 1.0, jnp.float32)

    def _zero_slice(i, _):
        slice_v[pl.ds(i * _LANES, _LANES)] = zero16
        return _

    def _drain_sweep(dst_sh):
        pltpu.make_async_copy(ubuf_v, dst_sh, sem).wait()

    lax.fori_loop(0, _SLICE // _LANES, _zero_slice, None)
    pltpu.sync_copy(slice_v, a_sh.at[pl.ds(base, _SLICE)])
    for k in range(8):
        vals_v[0, pl.ds(k * _LANES, _LANES)] = one16
    lax.fori_loop(0, nrows, _stage_drain, None)
    plsc.subcore_barrier()

    def _deg_row(j, _):
        pltpu.async_copy(vals_v.at[0], a_sh.at[dst_v.at[j]], sem, add=True)
        return _

    lax.fori_loop(0, _ROWS, _deg_row, None)
    _drain_sweep(a_sh)
    plsc.subcore_barrier()

    pltpu.sync_copy(a_sh.at[pl.ds(base, _SLICE)], slice_v)

    def _norm_step(i, _):
        d = slice_v[pl.ds(i * _LANES, _LANES)]
        slice_v[pl.ds(i * _LANES, _LANES)] = _rsqrt16(jnp.maximum(d, 1.0))
        return _

    lax.fori_loop(0, _SLICE // _LANES, _norm_step, None)
    pltpu.sync_copy(slice_v, b_sh.at[pl.ds(base, _SLICE)])
    plsc.subcore_barrier()

    pltpu.sync_copy(b_sh, ubuf_v)
    plsc.subcore_barrier()
    lax.fori_loop(0, _SLICE // _LANES, _zero_slice, None)
    pltpu.sync_copy(slice_v, b_sh.at[pl.ds(base, _SLICE)])
    plsc.subcore_barrier()

    inv_n = jnp.float32(1.0 / _N)

    def _w_row(j, _):
        for k in range(8):
            sl = pl.ds(k * _LANES, _LANES)
            ns = plsc.load_gather(ubuf_v, [src_v[j, sl]])
            nd = plsc.load_gather(ubuf_v, [dst_v[j, sl]])
            w = ns * nd
            w_v[j, sl] = w
            vals_v[j, sl] = w * inv_n
        pltpu.async_copy(vals_v.at[j], b_sh.at[src_v.at[j]], sem, add=True)
        return _

    lax.fori_loop(0, _ROWS, _w_row, None)
    _drain_sweep(b_sh)
    plsc.subcore_barrier()

    for r in range(1, _ROUNDS):
        rd, wr = (b_sh, a_sh) if r % 2 == 1 else (a_sh, b_sh)
        pltpu.sync_copy(rd, ubuf_v)
        lax.fori_loop(0, _SLICE // _LANES, _zero_slice, None)
        pltpu.sync_copy(slice_v, wr.at[pl.ds(base, _SLICE)])
        plsc.subcore_barrier()

        def _mp_row(j, _):
            for k in range(8):
                sl = pl.ds(k * _LANES, _LANES)
                uv = plsc.load_gather(ubuf_v, [dst_v[j, sl]])
                vals_v[j, sl] = uv * w_v[j, sl]
            pltpu.async_copy(vals_v.at[j], wr.at[src_v.at[j]], sem, add=True)
            return _

        lax.fori_loop(0, _ROWS, _mp_row, None)
        _drain_sweep(wr)
        plsc.subcore_barrier()

    pltpu.sync_copy(a_sh.at[pl.ds(base, _SLICE)], slice_v)

    @pl.when(c == 0)
    def _():
        pltpu.sync_copy(slice_v, out_hbm.at[0, pl.ds(base, _SLICE)])


_sc_u = functools.partial(
    pl.kernel,
    out_type=jax.ShapeDtypeStruct((1, _NP), jnp.float32),
    mesh=plsc.VectorSubcoreMesh(core_axis_name="c", subcore_axis_name="s"),
    compiler_params=pltpu.CompilerParams(needs_layout_passes=False),
    scratch_types=[
        pltpu.VMEM((_ROWS, 128), jnp.int32),
        pltpu.VMEM((_ROWS, 128), jnp.int32),
        pltpu.VMEM((_ROWS, 128), jnp.float32),
        pltpu.VMEM((_ROWS, 128), jnp.float32),
        pltpu.VMEM((_NP,), jnp.float32),
        pltpu.VMEM((_SLICE,), jnp.float32),
        pltpu.VMEM_SHARED((_NP,), jnp.float32),
        pltpu.VMEM_SHARED((_NP,), jnp.float32),
        pltpu.SemaphoreType.DMA,
    ],
)(_sc_body)


def _tc_body(u_ref, feat_ref, wemb_ref, bemb_ref, w1_ref, b1_ref, w2_ref,
             b2_ref, wr_ref, br_ref, out_ref):
    u = u_ref[...]
    dn = (((1,), (1,)), ((), ()))
    hi = lax.Precision.DEFAULT
    v1 = lax.dot_general(u, feat_ref[...], (((1,), (0,)), ((), ())),
                         preferred_element_type=jnp.float32, precision=hi)
    su = jnp.sum(u)
    hg = lax.dot_general(v1, wemb_ref[...], dn,
                         preferred_element_type=jnp.float32,
                         precision=hi) + su * bemb_ref[...].reshape(1, -1)
    hg = lax.dot_general(hg, w1_ref[...], dn,
                         preferred_element_type=jnp.float32,
                         precision=hi) + b1_ref[...].reshape(1, -1)
    hg = lax.dot_general(hg, w2_ref[...], dn,
                         preferred_element_type=jnp.float32,
                         precision=hi) + b2_ref[...].reshape(1, -1)
    out_ref[...] = lax.dot_general(hg, wr_ref[...], dn,
                                   preferred_element_type=jnp.float32,
                                   precision=hi) + br_ref[...].reshape(1, -1)


def kernel(feat, edge_index, e, snorm_n, snorm_e, W_emb, b_emb, W1, b1, W2,
           b2, Wr, br):
    u = _sc_u(edge_index)[:, :_N]
    nc = Wr.shape[0]
    out = pl.pallas_call(
        _tc_body,
        out_shape=jax.ShapeDtypeStruct((1, nc), jnp.float32),
    )(u, feat, W_emb, b_emb, W1, b1, W2, b2, Wr, br)
    return out
